# baseline (speedup 1.0000x reference)
import jax
import jax.numpy as jnp
from jax.experimental import pallas as pl
from jax.experimental.pallas import tpu as pltpu


def _make_gather(B, V, D, rows_per_blk=512, d_pad=2048):
    grid = B // rows_per_blk

    vh = 504
    vr = V - vh

    def body(labels_ref, t_lo_ref, t_hi_ref, out_ref, tab_al):
        @pl.when(pl.program_id(0) == 0)
        def _init():
            tab_al[:vh, :D] = t_lo_ref[...]
            tab_al[pl.ds(vh, vr), :D] = t_hi_ref[:vr, :]

        base = pl.program_id(0) * rows_per_blk
        for r in range(rows_per_blk):
            lbl = labels_ref[base + r]
            out_ref[pl.ds(r, 1), :] = tab_al[pl.ds(lbl, 1), :D]

    return pl.pallas_call(
        body,
        grid_spec=pltpu.PrefetchScalarGridSpec(
            num_scalar_prefetch=1,
            grid=(grid,),
            in_specs=[
                pl.BlockSpec((vh, D), lambda i, lref: (0, 0)),
                pl.BlockSpec((vh, D), lambda i, lref: (1, 0)),
            ],
            out_specs=pl.BlockSpec((rows_per_blk, D), lambda i, lref: (i, 0)),
            scratch_shapes=[pltpu.VMEM((V, d_pad), jnp.float32)],
        ),
        out_shape=jax.ShapeDtypeStruct((B, D), jnp.float32),
    )


def kernel(labels, table, train):
    del train
    B = labels.shape[0]
    V, D = table.shape
    k = _make_gather(B, V, D)
    labels32 = labels.astype(jnp.int32)
    return k(labels32, table, table)


# speedup vs baseline: 1.0034x; 1.0034x over previous
import jax
import jax.numpy as jnp
from jax.experimental import pallas as pl
from jax.experimental.pallas import tpu as pltpu


def _make_gather(B, V, D, rows_per_blk=512, d_pad=2048):
    grid = B // rows_per_blk

    def body(labels_ref, table_ref, out_ref, tab_al):
        @pl.when(pl.program_id(0) == 0)
        def _init():
            tab_al[:, :D] = table_ref[...]

        base = pl.program_id(0) * rows_per_blk
        for r in range(rows_per_blk):
            lbl = labels_ref[base + r]
            out_ref[pl.ds(r, 1), :] = tab_al[pl.ds(lbl, 1), :D]

    return pl.pallas_call(
        body,
        grid_spec=pltpu.PrefetchScalarGridSpec(
            num_scalar_prefetch=1,
            grid=(grid,),
            in_specs=[pl.BlockSpec((V, D), lambda i, lref: (0, 0))],
            out_specs=pl.BlockSpec((rows_per_blk, D), lambda i, lref: (i, 0)),
            scratch_shapes=[pltpu.VMEM((V, d_pad), jnp.float32)],
        ),
        out_shape=jax.ShapeDtypeStruct((B, D), jnp.float32),
    )


def kernel(labels, table, train):
    del train
    B = labels.shape[0]
    V, D = table.shape
    k = _make_gather(B, V, D)
    return k(labels.astype(jnp.int32), table)
